# baseline (speedup 1.0000x reference)
import functools

import jax
import jax.numpy as jnp
from jax import lax
from jax.experimental import pallas as pl
from jax.experimental.pallas import tpu as pltpu
from jax.experimental.pallas import tpu_sc as plsc

N_NODES = 100000
EMB_DIM = 64
D_FEAT = 128
D_OUT = EMB_DIM + D_FEAT

JC = 128
NUM_FULL = N_NODES // JC
TAIL = N_NODES - NUM_FULL * JC
NUM_CHUNKS = NUM_FULL + 1
NUM_WORKERS = 32
MAX_CHUNKS_PER_WORKER = -(-NUM_CHUNKS // NUM_WORKERS)

_MESH = plsc.VectorSubcoreMesh(core_axis_name="c", subcore_axis_name="s")


@functools.partial(
    pl.kernel,
    out_type=jax.ShapeDtypeStruct((D_OUT, N_NODES), jnp.float32),
    mesh=_MESH,
    scratch_types=[
        pltpu.VMEM((JC,), jnp.int32),
        pltpu.VMEM((JC,), jnp.int32),
        pltpu.VMEM((JC, 2 * EMB_DIM), jnp.float32),
        pltpu.VMEM((JC, D_FEAT), jnp.float32),
        pltpu.VMEM((D_OUT, JC), jnp.float32),
        pltpu.VMEM((D_OUT, TAIL), jnp.float32),
        pltpu.SemaphoreType.DMA,
    ],
    compiler_params=pltpu.CompilerParams(needs_layout_passes=False),
)
def _feature_prep_sc(
    ids_hbm, feats_hbm, table2_hbm, out_hbm,
    idx_v, idx2_v, pairs_v, feats_v, blk_v, blk_t, sem,
):
    wid = lax.axis_index("s") * _MESH.num_cores + lax.axis_index("c")
    lanes = jnp.arange(16, dtype=jnp.int32)

    def do_chunk(base, width, pairs_ref, idx2_ref, blk_ref):
        pltpu.sync_copy(ids_hbm.at[pl.ds(base, width)], idx_v.at[pl.ds(0, width)])
        for q in range(width // 16):
            idx2_v[pl.ds(q * 16, 16)] = idx_v[pl.ds(q * 16, 16)] >> 1
        gather = pltpu.async_copy(table2_hbm.at[idx2_ref], pairs_ref, sem)
        pltpu.sync_copy(
            feats_hbm.at[pl.ds(base, width)], feats_v.at[pl.ds(0, width)]
        )
        gather.wait()

        def col_group(g, carry):
            jj0 = g * 16
            row_idx = jj0 + lanes
            sel = (idx_v[pl.ds(jj0, 16)] & 1) * EMB_DIM

            def emb_rows(d8, c):
                for u in range(8):
                    d = d8 * 8 + u
                    val = plsc.load_gather(pairs_v, [row_idx, sel + d])
                    blk_ref[d, pl.ds(jj0, 16)] = val
                return c

            lax.fori_loop(0, EMB_DIM // 8, emb_rows, 0)

            def feat_rows(f8, c):
                for u in range(8):
                    f = f8 * 8 + u
                    val = plsc.load_gather(
                        feats_v, [row_idx, jnp.full((16,), 0, jnp.int32) + f]
                    )
                    blk_ref[EMB_DIM + f, pl.ds(jj0, 16)] = val
                return c

            lax.fori_loop(0, D_FEAT // 8, feat_rows, 0)
            return carry

        lax.fori_loop(0, width // 16, col_group, 0)
        pltpu.sync_copy(blk_ref, out_hbm.at[:, pl.ds(base, width)])

    for i in range(MAX_CHUNKS_PER_WORKER):
        chunk = wid + i * NUM_WORKERS

        @pl.when(chunk < NUM_FULL)
        def _():
            do_chunk(chunk * JC, JC, pairs_v, idx2_v, blk_v)

        @pl.when(chunk == NUM_FULL)
        def _():
            do_chunk(
                NUM_FULL * JC,
                TAIL,
                pairs_v.at[pl.ds(0, TAIL)],
                idx2_v.at[pl.ds(0, TAIL)],
                blk_t,
            )


def kernel(ids, feats, table):
    table2 = table.reshape(table.shape[0] // 2, 2 * table.shape[1])
    out_t = _feature_prep_sc(ids.astype(jnp.int32), feats, table2)
    return out_t.T


# speedup vs baseline: 1.2296x; 1.2296x over previous
import functools

import jax
import jax.numpy as jnp
from jax import lax
from jax.experimental import pallas as pl
from jax.experimental.pallas import tpu as pltpu
from jax.experimental.pallas import tpu_sc as plsc

N_NODES = 100000
EMB_DIM = 64
D_FEAT = 128
D_OUT = EMB_DIM + D_FEAT

JC = 128
NUM_FULL = N_NODES // JC
TAIL = N_NODES - NUM_FULL * JC
NUM_CHUNKS = NUM_FULL + 1
NUM_WORKERS = 32
MAX_CHUNKS_PER_WORKER = -(-NUM_CHUNKS // NUM_WORKERS)

_MESH = plsc.VectorSubcoreMesh(core_axis_name="c", subcore_axis_name="s")


@functools.partial(
    pl.kernel,
    out_type=jax.ShapeDtypeStruct((D_OUT, N_NODES), jnp.float32),
    mesh=_MESH,
    scratch_types=[
        pltpu.VMEM((JC,), jnp.int32),
        pltpu.VMEM((JC,), jnp.int32),
        pltpu.VMEM((JC, 2 * EMB_DIM), jnp.float32),
        pltpu.VMEM((JC, D_FEAT), jnp.float32),
        pltpu.VMEM((D_OUT, JC), jnp.float32),
        pltpu.VMEM((D_OUT, TAIL), jnp.float32),
        pltpu.SemaphoreType.DMA,
    ],
    compiler_params=pltpu.CompilerParams(needs_layout_passes=False),
)
def _feature_prep_sc(
    ids_hbm, feats_hbm, table2_hbm, out_hbm,
    idx_v, idx2_v, pairs_v, feats_v, blk_v, blk_t, sem,
):
    wid = lax.axis_index("s") * _MESH.num_cores + lax.axis_index("c")
    lanes = jnp.arange(16, dtype=jnp.int32)

    def do_chunk(base, width, pairs_ref, idx2_ref, blk_ref):
        pltpu.sync_copy(ids_hbm.at[pl.ds(base, width)], idx_v.at[pl.ds(0, width)])
        for q in range(width // 16):
            idx2_v[pl.ds(q * 16, 16)] = idx_v[pl.ds(q * 16, 16)] >> 1
        gather = pltpu.async_copy(table2_hbm.at[idx2_ref], pairs_ref, sem)
        pltpu.sync_copy(
            feats_hbm.at[pl.ds(base, width)], feats_v.at[pl.ds(0, width)]
        )
        gather.wait()

        for g in range(width // 16):
            jj0 = g * 16
            row_idx = jj0 + lanes
            sel = (idx_v[pl.ds(jj0, 16)] & 1) * EMB_DIM

            @plsc.parallel_loop(0, EMB_DIM, 1, unroll=8)
            def emb_rows(d):
                val = plsc.load_gather(pairs_v, [row_idx, sel + d])
                blk_ref[d, pl.ds(jj0, 16)] = val

            @plsc.parallel_loop(0, D_FEAT, 1, unroll=8)
            def feat_rows(f):
                val = plsc.load_gather(
                    feats_v, [row_idx, jnp.full((16,), 0, jnp.int32) + f]
                )
                blk_ref[EMB_DIM + f, pl.ds(jj0, 16)] = val

        pltpu.sync_copy(blk_ref, out_hbm.at[:, pl.ds(base, width)])

    def chunk_body(i, carry):
        chunk = wid + i * NUM_WORKERS

        @pl.when(chunk < NUM_FULL)
        def _():
            do_chunk(chunk * JC, JC, pairs_v, idx2_v, blk_v)

        @pl.when(chunk == NUM_FULL)
        def _():
            do_chunk(
                NUM_FULL * JC,
                TAIL,
                pairs_v.at[pl.ds(0, TAIL)],
                idx2_v.at[pl.ds(0, TAIL)],
                blk_t,
            )

        return carry

    lax.fori_loop(0, MAX_CHUNKS_PER_WORKER, chunk_body, 0)


def kernel(ids, feats, table):
    table2 = table.reshape(table.shape[0] // 2, 2 * table.shape[1])
    out_t = _feature_prep_sc(ids.astype(jnp.int32), feats, table2)
    return out_t.T


# speedup vs baseline: 1.3116x; 1.0667x over previous
import functools

import jax
import jax.numpy as jnp
from jax import lax
from jax.experimental import pallas as pl
from jax.experimental.pallas import tpu as pltpu
from jax.experimental.pallas import tpu_sc as plsc

N_NODES = 100000
EMB_DIM = 64
D_FEAT = 128
D_OUT = EMB_DIM + D_FEAT

JC = 128
NUM_FULL = N_NODES // JC
TAIL = N_NODES - NUM_FULL * JC
TAIL_BASE = NUM_FULL * JC
NUM_WORKERS = 32
TAIL_WID = NUM_FULL % NUM_WORKERS

_MESH = plsc.VectorSubcoreMesh(core_axis_name="c", subcore_axis_name="s")


@functools.partial(
    pl.kernel,
    out_type=jax.ShapeDtypeStruct((D_OUT, N_NODES), jnp.float32),
    mesh=_MESH,
    scratch_types=[
        pltpu.VMEM((2, JC), jnp.int32),
        pltpu.VMEM((2, JC), jnp.int32),
        pltpu.VMEM((2, JC, 2 * EMB_DIM), jnp.float32),
        pltpu.VMEM((2, JC, D_FEAT), jnp.float32),
        pltpu.VMEM((D_OUT, JC), jnp.float32),
        pltpu.VMEM((D_OUT, TAIL), jnp.float32),
        pltpu.SemaphoreType.DMA((2,)),
        pltpu.SemaphoreType.DMA((2,)),
        pltpu.SemaphoreType.DMA((2,)),
        pltpu.SemaphoreType.DMA,
    ],
    compiler_params=pltpu.CompilerParams(needs_layout_passes=False),
)
def _feature_prep_sc(
    ids_hbm, feats_hbm, table2_hbm, out_hbm,
    idx_v, idx2_v, pairs_v, feats_v, blk_v, blk_t,
    sem_i, sem_g, sem_f, sem_o,
):
    wid = lax.axis_index("s") * _MESH.num_cores + lax.axis_index("c")
    lanes = jnp.arange(16, dtype=jnp.int32)
    n_i = 24 + (wid < NUM_FULL - 24 * NUM_WORKERS).astype(jnp.int32)

    def assemble(b, blk_ref, width):
        for g in range(width // 16):
            jj0 = g * 16
            row_idx = jj0 + lanes
            sel = (idx_v[b, pl.ds(jj0, 16)] & 1) * EMB_DIM

            @plsc.parallel_loop(0, EMB_DIM, 1, unroll=8)
            def emb_rows(d):
                val = plsc.load_gather(pairs_v.at[b], [row_idx, sel + d])
                blk_ref[d, pl.ds(jj0, 16)] = val

            @plsc.parallel_loop(0, D_FEAT, 1, unroll=8)
            def feat_rows(f):
                val = plsc.load_gather(
                    feats_v.at[b], [row_idx, jnp.full((16,), 0, jnp.int32) + f]
                )
                blk_ref[EMB_DIM + f, pl.ds(jj0, 16)] = val

    def launch(i):
        b = i & 1
        base = (wid + i * NUM_WORKERS) * JC
        pltpu.make_async_copy(
            ids_hbm.at[pl.ds(base, JC)], idx_v.at[b], sem_i.at[b]
        ).wait()
        for q in range(JC // 16):
            idx2_v[b, pl.ds(q * 16, 16)] = idx_v[b, pl.ds(q * 16, 16)] >> 1
        pltpu.async_copy(table2_hbm.at[idx2_v.at[b]], pairs_v.at[b], sem_g.at[b])
        pltpu.async_copy(feats_hbm.at[pl.ds(base, JC)], feats_v.at[b], sem_f.at[b])

    def retire(i):
        b = i & 1
        base = (wid + i * NUM_WORKERS) * JC

        @pl.when(i >= 1)
        def _():
            pbase = (wid + (i - 1) * NUM_WORKERS) * JC
            pltpu.make_async_copy(
                blk_v, out_hbm.at[:, pl.ds(pbase, JC)], sem_o
            ).wait()

        pltpu.make_async_copy(
            table2_hbm.at[idx2_v.at[b]], pairs_v.at[b], sem_g.at[b]
        ).wait()
        pltpu.make_async_copy(
            feats_hbm.at[pl.ds(base, JC)], feats_v.at[b], sem_f.at[b]
        ).wait()
        assemble(b, blk_v, JC)
        pltpu.async_copy(blk_v, out_hbm.at[:, pl.ds(base, JC)], sem_o)

    pltpu.async_copy(
        ids_hbm.at[pl.ds(wid * JC, JC)], idx_v.at[0], sem_i.at[0]
    )

    def body(i, carry):
        launch(i)

        @pl.when(i + 1 < n_i)
        def _():
            nb = (i + 1) & 1
            nbase = (wid + (i + 1) * NUM_WORKERS) * JC
            pltpu.async_copy(
                ids_hbm.at[pl.ds(nbase, JC)], idx_v.at[nb], sem_i.at[nb]
            )

        @pl.when(i >= 1)
        def _():
            retire(i - 1)

        return carry

    lax.fori_loop(0, n_i, body, 0)
    retire(n_i - 1)
    pltpu.make_async_copy(
        blk_v, out_hbm.at[:, pl.ds((wid + (n_i - 1) * NUM_WORKERS) * JC, JC)], sem_o
    ).wait()

    @pl.when(wid == TAIL_WID)
    def _():
        pltpu.sync_copy(
            ids_hbm.at[pl.ds(TAIL_BASE, TAIL)], idx_v.at[0].at[pl.ds(0, TAIL)]
        )
        for q in range(TAIL // 16):
            idx2_v[0, pl.ds(q * 16, 16)] = idx_v[0, pl.ds(q * 16, 16)] >> 1
        pltpu.async_copy(
            table2_hbm.at[idx2_v.at[0].at[pl.ds(0, TAIL)]],
            pairs_v.at[0].at[pl.ds(0, TAIL)],
            sem_g.at[0],
        ).wait()
        pltpu.sync_copy(
            feats_hbm.at[pl.ds(TAIL_BASE, TAIL)],
            feats_v.at[0].at[pl.ds(0, TAIL)],
        )
        assemble(0, blk_t, TAIL)
        pltpu.sync_copy(blk_t, out_hbm.at[:, pl.ds(TAIL_BASE, TAIL)])


def kernel(ids, feats, table):
    table2 = table.reshape(table.shape[0] // 2, 2 * table.shape[1])
    out_t = _feature_prep_sc(ids.astype(jnp.int32), feats, table2)
    return out_t.T


# speedup vs baseline: 1.3137x; 1.0016x over previous
import functools

import jax
import jax.numpy as jnp
from jax import lax
from jax.experimental import pallas as pl
from jax.experimental.pallas import tpu as pltpu
from jax.experimental.pallas import tpu_sc as plsc

N_NODES = 100000
EMB_DIM = 64
D_FEAT = 128
D_OUT = EMB_DIM + D_FEAT

JC = 128
NUM_FULL = N_NODES // JC
TAIL = N_NODES - NUM_FULL * JC
TAIL_BASE = NUM_FULL * JC
NUM_WORKERS = 32
TAIL_WID = NUM_FULL % NUM_WORKERS

_MESH = plsc.VectorSubcoreMesh(core_axis_name="c", subcore_axis_name="s")


@functools.partial(
    pl.kernel,
    out_type=jax.ShapeDtypeStruct((D_OUT, N_NODES), jnp.float32),
    mesh=_MESH,
    scratch_types=[
        pltpu.VMEM((3, JC), jnp.int32),
        pltpu.VMEM((2, JC), jnp.int32),
        pltpu.VMEM((2, JC, 2 * EMB_DIM), jnp.float32),
        pltpu.VMEM((2, JC, D_FEAT), jnp.float32),
        pltpu.VMEM((D_OUT, JC), jnp.float32),
        pltpu.VMEM((D_OUT, TAIL), jnp.float32),
        pltpu.SemaphoreType.DMA((3,)),
        pltpu.SemaphoreType.DMA((2,)),
        pltpu.SemaphoreType.DMA((2,)),
        pltpu.SemaphoreType.DMA,
    ],
    compiler_params=pltpu.CompilerParams(needs_layout_passes=False),
)
def _feature_prep_sc(
    ids_hbm, feats_hbm, table2_hbm, out_hbm,
    idx_v, idx2_v, pairs_v, feats_v, blk_v, blk_t,
    sem_i, sem_g, sem_f, sem_o,
):
    wid = lax.axis_index("s") * _MESH.num_cores + lax.axis_index("c")
    lanes = jnp.arange(16, dtype=jnp.int32)
    n_i = 24 + (wid < NUM_FULL - 24 * NUM_WORKERS).astype(jnp.int32)

    def assemble(bi, b, blk_ref, width):
        for g in range(width // 16):
            jj0 = g * 16
            row_idx = jj0 + lanes
            sel = (idx_v[bi, pl.ds(jj0, 16)] & 1) * EMB_DIM

            @plsc.parallel_loop(0, EMB_DIM, 1, unroll=8)
            def emb_rows(d):
                val = plsc.load_gather(pairs_v.at[b], [row_idx, sel + d])
                blk_ref[d, pl.ds(jj0, 16)] = val

            @plsc.parallel_loop(0, D_FEAT, 1, unroll=8)
            def feat_rows(f):
                val = plsc.load_gather(
                    feats_v.at[b], [row_idx, jnp.full((16,), 0, jnp.int32) + f]
                )
                blk_ref[EMB_DIM + f, pl.ds(jj0, 16)] = val

    def launch(i):
        b = i & 1
        bi = i % 3
        base = (wid + i * NUM_WORKERS) * JC
        pltpu.make_async_copy(
            ids_hbm.at[pl.ds(base, JC)], idx_v.at[bi], sem_i.at[bi]
        ).wait()
        for q in range(JC // 16):
            idx2_v[b, pl.ds(q * 16, 16)] = idx_v[bi, pl.ds(q * 16, 16)] >> 1
        pltpu.async_copy(table2_hbm.at[idx2_v.at[b]], pairs_v.at[b], sem_g.at[b])
        pltpu.async_copy(feats_hbm.at[pl.ds(base, JC)], feats_v.at[b], sem_f.at[b])

    def retire(i):
        b = i & 1
        base = (wid + i * NUM_WORKERS) * JC

        @pl.when(i >= 1)
        def _():
            pbase = (wid + (i - 1) * NUM_WORKERS) * JC
            pltpu.make_async_copy(
                blk_v, out_hbm.at[:, pl.ds(pbase, JC)], sem_o
            ).wait()

        pltpu.make_async_copy(
            table2_hbm.at[idx2_v.at[b]], pairs_v.at[b], sem_g.at[b]
        ).wait()
        pltpu.make_async_copy(
            feats_hbm.at[pl.ds(base, JC)], feats_v.at[b], sem_f.at[b]
        ).wait()
        assemble(i % 3, b, blk_v, JC)
        pltpu.async_copy(blk_v, out_hbm.at[:, pl.ds(base, JC)], sem_o)

    pltpu.async_copy(
        ids_hbm.at[pl.ds(wid * JC, JC)], idx_v.at[0], sem_i.at[0]
    )

    def body(i, carry):
        launch(i)

        @pl.when(i + 1 < n_i)
        def _():
            nb = (i + 1) % 3
            nbase = (wid + (i + 1) * NUM_WORKERS) * JC
            pltpu.async_copy(
                ids_hbm.at[pl.ds(nbase, JC)], idx_v.at[nb], sem_i.at[nb]
            )

        @pl.when(i >= 1)
        def _():
            retire(i - 1)

        return carry

    lax.fori_loop(0, n_i, body, 0)
    retire(n_i - 1)
    pltpu.make_async_copy(
        blk_v, out_hbm.at[:, pl.ds((wid + (n_i - 1) * NUM_WORKERS) * JC, JC)], sem_o
    ).wait()

    @pl.when(wid == TAIL_WID)
    def _():
        pltpu.sync_copy(
            ids_hbm.at[pl.ds(TAIL_BASE, TAIL)], idx_v.at[0].at[pl.ds(0, TAIL)]
        )
        for q in range(TAIL // 16):
            idx2_v[0, pl.ds(q * 16, 16)] = idx_v[0, pl.ds(q * 16, 16)] >> 1
        pltpu.async_copy(
            table2_hbm.at[idx2_v.at[0].at[pl.ds(0, TAIL)]],
            pairs_v.at[0].at[pl.ds(0, TAIL)],
            sem_g.at[0],
        ).wait()
        pltpu.sync_copy(
            feats_hbm.at[pl.ds(TAIL_BASE, TAIL)],
            feats_v.at[0].at[pl.ds(0, TAIL)],
        )
        assemble(0, 0, blk_t, TAIL)
        pltpu.sync_copy(blk_t, out_hbm.at[:, pl.ds(TAIL_BASE, TAIL)])


def kernel(ids, feats, table):
    table2 = table.reshape(table.shape[0] // 2, 2 * table.shape[1])
    out_t = _feature_prep_sc(ids.astype(jnp.int32), feats, table2)
    return out_t.T


# speedup vs baseline: 1.5774x; 1.2007x over previous
import functools

import jax
import jax.numpy as jnp
from jax import lax
from jax.experimental import pallas as pl
from jax.experimental.pallas import tpu as pltpu
from jax.experimental.pallas import tpu_sc as plsc

N_NODES = 100000
EMB_DIM = 64
D_FEAT = 128
D_OUT = EMB_DIM + D_FEAT

JC = 128
NUM_FULL = N_NODES // JC
TAIL = N_NODES - NUM_FULL * JC
TAIL_BASE = NUM_FULL * JC
NUM_WORKERS = 32
TAIL_WID = NUM_FULL % NUM_WORKERS

_MESH = plsc.VectorSubcoreMesh(core_axis_name="c", subcore_axis_name="s")


@functools.partial(
    pl.kernel,
    out_type=jax.ShapeDtypeStruct((D_OUT, N_NODES), jnp.float32),
    mesh=_MESH,
    scratch_types=[
        pltpu.VMEM((3, JC), jnp.int32),
        pltpu.VMEM((2, JC), jnp.int32),
        pltpu.VMEM((2, JC, 2 * EMB_DIM), jnp.float32),
        pltpu.VMEM((2, JC, D_FEAT), jnp.float32),
        pltpu.VMEM((D_OUT, JC), jnp.float32),
        pltpu.VMEM((D_OUT, TAIL), jnp.float32),
        pltpu.VMEM((32, 256), jnp.float32),
        pltpu.SemaphoreType.DMA((3,)),
        pltpu.SemaphoreType.DMA((2,)),
        pltpu.SemaphoreType.DMA((2,)),
        pltpu.SemaphoreType.DMA,
    ],
    compiler_params=pltpu.CompilerParams(needs_layout_passes=False),
)
def _feature_prep_sc(
    ids_hbm, feats_hbm, table2_hbm, out_hbm,
    idx_v, idx2_v, pairs_v, feats_v, blk_v, blk_t, skew_v,
    sem_i, sem_g, sem_f, sem_o,
):
    wid = lax.axis_index("s") * _MESH.num_cores + lax.axis_index("c")
    lanes = jnp.arange(16, dtype=jnp.int32)
    n_i = 24 + (wid < NUM_FULL - 24 * NUM_WORKERS).astype(jnp.int32)

    def assemble(bi, b, blk_ref, width):
        for q in range(width // 32):
            q32 = q * 32

            def fill(src_ref):
                @plsc.parallel_loop(0, 32, 1, unroll=2)
                def _(r):
                    for c in range(8):
                        skew_v[r, pl.ds(r + 16 * c, 16)] = src_ref[
                            q32 + r, pl.ds(16 * c, 16)
                        ]

            fill(pairs_v.at[b])
            for g in range(2):
                rowloc = g * 16 + lanes
                jj0 = q32 + g * 16
                sel = (idx_v[bi, pl.ds(jj0, 16)] & 1) * EMB_DIM
                colb = rowloc + sel

                @plsc.parallel_loop(0, EMB_DIM, 1, unroll=8)
                def emb_rows(d):
                    val = plsc.load_gather(skew_v, [rowloc, colb + d])
                    blk_ref[d, pl.ds(jj0, 16)] = val

            fill(feats_v.at[b])
            for g in range(2):
                rowloc = g * 16 + lanes
                jj0 = q32 + g * 16

                @plsc.parallel_loop(0, D_FEAT, 1, unroll=8)
                def feat_rows(f):
                    val = plsc.load_gather(skew_v, [rowloc, rowloc + f])
                    blk_ref[EMB_DIM + f, pl.ds(jj0, 16)] = val

    def launch(i):
        b = i & 1
        bi = i % 3
        base = (wid + i * NUM_WORKERS) * JC
        pltpu.make_async_copy(
            ids_hbm.at[pl.ds(base, JC)], idx_v.at[bi], sem_i.at[bi]
        ).wait()
        for q in range(JC // 16):
            idx2_v[b, pl.ds(q * 16, 16)] = idx_v[bi, pl.ds(q * 16, 16)] >> 1
        pltpu.async_copy(table2_hbm.at[idx2_v.at[b]], pairs_v.at[b], sem_g.at[b])
        pltpu.async_copy(feats_hbm.at[pl.ds(base, JC)], feats_v.at[b], sem_f.at[b])

    def retire(i):
        b = i & 1
        base = (wid + i * NUM_WORKERS) * JC

        @pl.when(i >= 1)
        def _():
            pbase = (wid + (i - 1) * NUM_WORKERS) * JC
            pltpu.make_async_copy(
                blk_v, out_hbm.at[:, pl.ds(pbase, JC)], sem_o
            ).wait()

        pltpu.make_async_copy(
            table2_hbm.at[idx2_v.at[b]], pairs_v.at[b], sem_g.at[b]
        ).wait()
        pltpu.make_async_copy(
            feats_hbm.at[pl.ds(base, JC)], feats_v.at[b], sem_f.at[b]
        ).wait()
        assemble(i % 3, b, blk_v, JC)
        pltpu.async_copy(blk_v, out_hbm.at[:, pl.ds(base, JC)], sem_o)

    pltpu.async_copy(
        ids_hbm.at[pl.ds(wid * JC, JC)], idx_v.at[0], sem_i.at[0]
    )

    def body(i, carry):
        launch(i)

        @pl.when(i + 1 < n_i)
        def _():
            nb = (i + 1) % 3
            nbase = (wid + (i + 1) * NUM_WORKERS) * JC
            pltpu.async_copy(
                ids_hbm.at[pl.ds(nbase, JC)], idx_v.at[nb], sem_i.at[nb]
            )

        @pl.when(i >= 1)
        def _():
            retire(i - 1)

        return carry

    lax.fori_loop(0, n_i, body, 0)
    retire(n_i - 1)
    pltpu.make_async_copy(
        blk_v, out_hbm.at[:, pl.ds((wid + (n_i - 1) * NUM_WORKERS) * JC, JC)], sem_o
    ).wait()

    @pl.when(wid == TAIL_WID)
    def _():
        pltpu.sync_copy(
            ids_hbm.at[pl.ds(TAIL_BASE, TAIL)], idx_v.at[0].at[pl.ds(0, TAIL)]
        )
        for q in range(TAIL // 16):
            idx2_v[0, pl.ds(q * 16, 16)] = idx_v[0, pl.ds(q * 16, 16)] >> 1
        pltpu.async_copy(
            table2_hbm.at[idx2_v.at[0].at[pl.ds(0, TAIL)]],
            pairs_v.at[0].at[pl.ds(0, TAIL)],
            sem_g.at[0],
        ).wait()
        pltpu.sync_copy(
            feats_hbm.at[pl.ds(TAIL_BASE, TAIL)],
            feats_v.at[0].at[pl.ds(0, TAIL)],
        )
        assemble(0, 0, blk_t, TAIL)
        pltpu.sync_copy(blk_t, out_hbm.at[:, pl.ds(TAIL_BASE, TAIL)])


def kernel(ids, feats, table):
    table2 = table.reshape(table.shape[0] // 2, 2 * table.shape[1])
    out_t = _feature_prep_sc(ids.astype(jnp.int32), feats, table2)
    return out_t.T
